# Initial kernel scaffold; baseline (speedup 1.0000x reference)
import jax
import jax.numpy as jnp
from jax.experimental import pallas as pl


def kernel(node_feat, edge_index, edge_feat, params):
    raise NotImplementedError("write your pallas kernel here")


# speedup vs baseline: 3.4926x; 3.4926x over previous
import functools

import jax
import jax.numpy as jnp
from jax import lax
from jax.experimental import pallas as pl
from jax.experimental.pallas import tpu as pltpu
from jax.experimental.pallas import tpu_sc as plsc

F32 = jnp.float32

N_NODES = 10000
N_EDGES = 320000
D_FEAT = 128
HID = 128
D_EDGE = 16
NUM_CONV = 4
NUM_DENSE = 3
EPS = 1e-3

NW = 32
CHUNK = 128
CHUNKS_PER_W = (N_EDGES + NW * CHUNK - 1) // (NW * CHUNK)
EDGES_PER_W = CHUNKS_PER_W * CHUNK
EP = NW * EDGES_PER_W
NP = 10112
ROWS_PER_TILE = NP // 16

_sc_mesh = plsc.VectorSubcoreMesh(core_axis_name="c", subcore_axis_name="s")


@functools.partial(
    pl.kernel,
    out_type=jax.ShapeDtypeStruct((2, NP, HID), F32),
    mesh=_sc_mesh,
    scratch_types=[
        pltpu.VMEM((CHUNKS_PER_W, CHUNK), jnp.int32),
        pltpu.VMEM((CHUNK, HID), F32),
        pltpu.VMEM_SHARED((NP, HID), F32),
    ],
)
def _sc_edge_scatter(ef_hbm, rowp_hbm, zeros_hbm, out_hbm,
                     row_slab, dbuf, acc):
    c = lax.axis_index("c")
    s = lax.axis_index("s")
    w = c * 16 + s
    pltpu.sync_copy(rowp_hbm.at[w], row_slab)
    pltpu.sync_copy(zeros_hbm, acc.at[pl.ds(s * ROWS_PER_TILE, ROWS_PER_TILE)])
    plsc.subcore_barrier()

    def body(j, _):
        pltpu.sync_copy(ef_hbm.at[pl.ds(w * EDGES_PER_W + j * CHUNK, CHUNK)], dbuf)
        pltpu.sync_copy(dbuf, acc.at[row_slab.at[j]], add=True)
        return _

    lax.fori_loop(0, CHUNKS_PER_W, body, None)
    plsc.subcore_barrier()
    pltpu.sync_copy(acc.at[pl.ds(s * ROWS_PER_TILE, ROWS_PER_TILE)],
                    out_hbm.at[c, pl.ds(s * ROWS_PER_TILE, ROWS_PER_TILE)])


@functools.partial(
    pl.kernel,
    out_type=jax.ShapeDtypeStruct((2, NP, HID), F32),
    mesh=_sc_mesh,
    scratch_types=[
        pltpu.VMEM((CHUNKS_PER_W, CHUNK), jnp.int32),
        pltpu.VMEM((CHUNKS_PER_W, CHUNK), jnp.int32),
        pltpu.VMEM((CHUNK, HID), F32),
        pltpu.VMEM_SHARED((NP, HID), F32),
        pltpu.SemaphoreType.DMA,
    ],
)
def _sc_gather_scatter(x_hbm, colp_hbm, rowp_hbm, zeros_hbm, out_hbm,
                       col_slab, row_slab, gbuf, acc, sem):
    c = lax.axis_index("c")
    s = lax.axis_index("s")
    w = c * 16 + s
    pltpu.sync_copy(colp_hbm.at[w], col_slab)
    pltpu.sync_copy(rowp_hbm.at[w], row_slab)
    pltpu.sync_copy(zeros_hbm, acc.at[pl.ds(s * ROWS_PER_TILE, ROWS_PER_TILE)])
    plsc.subcore_barrier()

    def body(j, _):
        pltpu.async_copy(x_hbm.at[col_slab.at[j]], gbuf, sem).wait()
        pltpu.sync_copy(gbuf, acc.at[row_slab.at[j]], add=True)
        return _

    lax.fori_loop(0, CHUNKS_PER_W, body, None)
    plsc.subcore_barrier()
    pltpu.sync_copy(acc.at[pl.ds(s * ROWS_PER_TILE, ROWS_PER_TILE)],
                    out_hbm.at[c, pl.ds(s * ROWS_PER_TILE, ROWS_PER_TILE)])


_BLK = 1264
_GRID = NP // _BLK


def _proj_body(nf_ref, wn_ref, bn_ref, ef2_ref, we_ref, x0_ref, eagg_ref):
    x0_ref[...] = (jnp.dot(nf_ref[...], wn_ref[...], preferred_element_type=F32, precision=jax.lax.Precision.HIGHEST)
                   + bn_ref[...])
    efs = ef2_ref[0] + ef2_ref[1]
    eagg_ref[...] = jnp.dot(efs, we_ref[...], preferred_element_type=F32, precision=jax.lax.Precision.HIGHEST)


def _tc_project(nf_pad, wn, bn, ef2, we_aug):
    return pl.pallas_call(
        _proj_body,
        grid=(_GRID,),
        in_specs=[
            pl.BlockSpec((_BLK, D_FEAT), lambda i: (i, 0)),
            pl.BlockSpec((D_FEAT, HID), lambda i: (0, 0)),
            pl.BlockSpec((1, HID), lambda i: (0, 0)),
            pl.BlockSpec((2, _BLK, HID), lambda i: (0, i, 0)),
            pl.BlockSpec((HID, HID), lambda i: (0, 0)),
        ],
        out_specs=[
            pl.BlockSpec((_BLK, HID), lambda i: (i, 0)),
            pl.BlockSpec((_BLK, HID), lambda i: (i, 0)),
        ],
        out_shape=[
            jax.ShapeDtypeStruct((NP, HID), F32),
            jax.ShapeDtypeStruct((NP, HID), F32),
        ],
    )(nf_pad, wn, bn, ef2, we_aug)


def _layer_body(g2_ref, e_ref, x_ref, w_ref, b_ref, sc_ref, sh_ref, out_ref):
    g = g2_ref[0] + g2_ref[1] + e_ref[...]
    h = jnp.dot(g, w_ref[...], preferred_element_type=F32, precision=jax.lax.Precision.HIGHEST) + b_ref[...]
    out_ref[...] = x_ref[...] + jnp.maximum(h, 0.0) * sc_ref[...] + sh_ref[...]


def _tc_layer(g2, eagg, x, w, b, scale, shift):
    return pl.pallas_call(
        _layer_body,
        grid=(_GRID,),
        in_specs=[
            pl.BlockSpec((2, _BLK, HID), lambda i: (0, i, 0)),
            pl.BlockSpec((_BLK, HID), lambda i: (i, 0)),
            pl.BlockSpec((_BLK, HID), lambda i: (i, 0)),
            pl.BlockSpec((HID, HID), lambda i: (0, 0)),
            pl.BlockSpec((1, HID), lambda i: (0, 0)),
            pl.BlockSpec((1, HID), lambda i: (0, 0)),
            pl.BlockSpec((1, HID), lambda i: (0, 0)),
        ],
        out_specs=pl.BlockSpec((_BLK, HID), lambda i: (i, 0)),
        out_shape=jax.ShapeDtypeStruct((NP, HID), F32),
    )(g2, eagg, x, w, b, scale, shift)


def _head_body(x_ref, wd_ref, bd_ref, sc_ref, sh_ref, wf_ref, bf_ref, out_ref):
    rows = lax.broadcasted_iota(jnp.int32, (NP, 1), 0)
    mask = (rows < N_NODES).astype(F32)
    m = jnp.sum(x_ref[...] * mask, axis=0, keepdims=True) / float(N_NODES)
    for i in range(NUM_DENSE):
        h = jnp.dot(m, wd_ref[i], preferred_element_type=F32, precision=jax.lax.Precision.HIGHEST) + bd_ref[i]
        m = jnp.maximum(h, 0.0) * sc_ref[i] + sh_ref[i]
    out_ref[...] = jnp.dot(m, wf_ref[...], preferred_element_type=F32, precision=jax.lax.Precision.HIGHEST) + bf_ref[...]


def _tc_head(x, wd, bd, scale, shift, wf, bf):
    return pl.pallas_call(
        _head_body,
        in_specs=[
            pl.BlockSpec((NP, HID), lambda: (0, 0)),
            pl.BlockSpec((NUM_DENSE, HID, HID), lambda: (0, 0, 0)),
            pl.BlockSpec((NUM_DENSE, 1, HID), lambda: (0, 0, 0)),
            pl.BlockSpec((NUM_DENSE, 1, HID), lambda: (0, 0, 0)),
            pl.BlockSpec((NUM_DENSE, 1, HID), lambda: (0, 0, 0)),
            pl.BlockSpec((HID, HID), lambda: (0, 0)),
            pl.BlockSpec((1, HID), lambda: (0, 0)),
        ],
        out_specs=pl.BlockSpec((1, HID), lambda: (0, 0)),
        out_shape=jax.ShapeDtypeStruct((1, HID), F32),
    )(x, wd, bd, scale, shift, wf, bf)


def kernel(node_feat, edge_index, edge_feat, params):
    p = params
    row = edge_index[0]
    col = edge_index[1]
    pad = EP - N_EDGES
    rowp = jnp.concatenate([row, jnp.full((pad,), N_NODES, jnp.int32)])
    colp = jnp.concatenate([col, jnp.zeros((pad,), jnp.int32)])
    rowp = rowp.reshape(NW, CHUNKS_PER_W, CHUNK)
    colp = colp.reshape(NW, CHUNKS_PER_W, CHUNK)

    ef_aug = jnp.concatenate(
        [edge_feat, jnp.ones((N_EDGES, 1), F32),
         jnp.zeros((N_EDGES, HID - 17), F32)], 1)
    ef_aug = jnp.concatenate([ef_aug, jnp.zeros((pad, HID), F32)], 0)

    nf_pad = jnp.concatenate([node_feat, jnp.zeros((NP - N_NODES, D_FEAT), F32)], 0)
    zeros128 = jnp.zeros((ROWS_PER_TILE, HID), F32)

    we_aug = jnp.concatenate(
        [p["W_edge"], p["b_edge"].reshape(1, HID), jnp.zeros((HID - 17, HID), F32)], 0)

    ef2 = _sc_edge_scatter(ef_aug, rowp, zeros128)
    x, eagg = _tc_project(nf_pad, p["W_node"], p["b_node"].reshape(1, HID),
                          ef2, we_aug)

    for i in range(NUM_CONV):
        g2 = _sc_gather_scatter(x, colp, rowp, zeros128)
        scale = p["g_conv%d" % i] * lax.rsqrt(p["mv_conv%d" % i] + EPS)
        shift = p["be_conv%d" % i] - p["mm_conv%d" % i] * scale
        x = _tc_layer(g2, eagg, x, p["W_conv%d" % i],
                      p["b_conv%d" % i].reshape(1, HID),
                      scale.reshape(1, HID), shift.reshape(1, HID))

    wd = jnp.stack([p["W_out%d" % i] for i in range(NUM_DENSE)])
    bd = jnp.stack([p["b_out%d" % i].reshape(1, HID) for i in range(NUM_DENSE)])
    scs, shs = [], []
    for i in range(NUM_DENSE):
        sc = p["g_out%d" % i] * lax.rsqrt(p["mv_out%d" % i] + EPS)
        scs.append(sc.reshape(1, HID))
        shs.append((p["be_out%d" % i] - p["mm_out%d" % i] * sc).reshape(1, HID))
    wf = jnp.concatenate([p["W_final"], jnp.zeros((HID, HID - 3), F32)], 1)
    bf = jnp.concatenate([p["b_final"], jnp.zeros((HID - 3,), F32)]).reshape(1, HID)

    out = _tc_head(x, wd, bd, jnp.stack(scs), jnp.stack(shs), wf, bf)
    return out[0, :3]


# speedup vs baseline: 3.6510x; 1.0454x over previous
import functools

import jax
import jax.numpy as jnp
from jax import lax
from jax.experimental import pallas as pl
from jax.experimental.pallas import tpu as pltpu
from jax.experimental.pallas import tpu_sc as plsc

F32 = jnp.float32

N_NODES = 10000
N_EDGES = 320000
D_FEAT = 128
HID = 128
D_EDGE = 16
NUM_CONV = 4
NUM_DENSE = 3
EPS = 1e-3

NW = 32
CHUNK = 128
CHUNKS_PER_W = (N_EDGES + NW * CHUNK - 1) // (NW * CHUNK)
EDGES_PER_W = CHUNKS_PER_W * CHUNK
EP = NW * EDGES_PER_W
NP = 10112
ROWS_PER_TILE = NP // 16

_sc_mesh = plsc.VectorSubcoreMesh(core_axis_name="c", subcore_axis_name="s")


@functools.partial(
    pl.kernel,
    out_type=jax.ShapeDtypeStruct((2, NP, HID), F32),
    mesh=_sc_mesh,
    scratch_types=[
        pltpu.VMEM((CHUNKS_PER_W, CHUNK), jnp.int32),
        pltpu.VMEM((CHUNK, HID), F32),
        pltpu.VMEM_SHARED((NP, HID), F32),
    ],
)
def _sc_edge_scatter(ef_hbm, rowp_hbm, zeros_hbm, out_hbm,
                     row_slab, dbuf, acc):
    c = lax.axis_index("c")
    s = lax.axis_index("s")
    w = c * 16 + s
    pltpu.sync_copy(rowp_hbm.at[w], row_slab)
    pltpu.sync_copy(zeros_hbm, acc.at[pl.ds(s * ROWS_PER_TILE, ROWS_PER_TILE)])
    plsc.subcore_barrier()

    def body(j, _):
        pltpu.sync_copy(ef_hbm.at[pl.ds(w * EDGES_PER_W + j * CHUNK, CHUNK)], dbuf)
        pltpu.sync_copy(dbuf, acc.at[row_slab.at[j]], add=True)
        return _

    lax.fori_loop(0, CHUNKS_PER_W, body, None)
    plsc.subcore_barrier()
    pltpu.sync_copy(acc.at[pl.ds(s * ROWS_PER_TILE, ROWS_PER_TILE)],
                    out_hbm.at[c, pl.ds(s * ROWS_PER_TILE, ROWS_PER_TILE)])


@functools.partial(
    pl.kernel,
    out_type=jax.ShapeDtypeStruct((2, NP, HID), F32),
    mesh=_sc_mesh,
    scratch_types=[
        pltpu.VMEM((CHUNKS_PER_W, CHUNK), jnp.int32),
        pltpu.VMEM((CHUNKS_PER_W, CHUNK), jnp.int32),
        pltpu.VMEM((CHUNK, HID), F32),
        pltpu.VMEM_SHARED((NP, HID), F32),
        pltpu.SemaphoreType.DMA,
    ],
)
def _sc_gather_scatter(x_hbm, colp_hbm, rowp_hbm, zeros_hbm, out_hbm,
                       col_slab, row_slab, gbuf, acc, sem):
    c = lax.axis_index("c")
    s = lax.axis_index("s")
    w = c * 16 + s
    pltpu.sync_copy(colp_hbm.at[w], col_slab)
    pltpu.sync_copy(rowp_hbm.at[w], row_slab)
    pltpu.sync_copy(zeros_hbm, acc.at[pl.ds(s * ROWS_PER_TILE, ROWS_PER_TILE)])
    plsc.subcore_barrier()

    def body(j, _):
        pltpu.async_copy(x_hbm.at[col_slab.at[j]], gbuf, sem).wait()
        pltpu.sync_copy(gbuf, acc.at[row_slab.at[j]], add=True)
        return _

    lax.fori_loop(0, CHUNKS_PER_W, body, None)
    plsc.subcore_barrier()
    pltpu.sync_copy(acc.at[pl.ds(s * ROWS_PER_TILE, ROWS_PER_TILE)],
                    out_hbm.at[c, pl.ds(s * ROWS_PER_TILE, ROWS_PER_TILE)])


_BLK = 1264
_GRID = NP // _BLK


def _proj_body(nf_ref, wn_ref, bn_ref, ef2_ref, x0_ref, eagg_ref):
    x0_ref[...] = (jnp.dot(nf_ref[...], wn_ref[...], preferred_element_type=F32)
                   + bn_ref[...])
    eagg_ref[...] = ef2_ref[0] + ef2_ref[1]


def _tc_project(nf_pad, wn, bn, ef2):
    return pl.pallas_call(
        _proj_body,
        grid=(_GRID,),
        in_specs=[
            pl.BlockSpec((_BLK, D_FEAT), lambda i: (i, 0)),
            pl.BlockSpec((D_FEAT, HID), lambda i: (0, 0)),
            pl.BlockSpec((1, HID), lambda i: (0, 0)),
            pl.BlockSpec((2, _BLK, HID), lambda i: (0, i, 0)),
        ],
        out_specs=[
            pl.BlockSpec((_BLK, HID), lambda i: (i, 0)),
            pl.BlockSpec((_BLK, HID), lambda i: (i, 0)),
        ],
        out_shape=[
            jax.ShapeDtypeStruct((NP, HID), F32),
            jax.ShapeDtypeStruct((NP, HID), F32),
        ],
    )(nf_pad, wn, bn, ef2)


_EBLK = 4096
_EGRID = EP // _EBLK


def _ea_body(ef_ref, we_ref, be_ref, out_ref):
    out_ref[...] = (jnp.dot(ef_ref[...], we_ref[...], preferred_element_type=F32)
                    + be_ref[...])


def _tc_edge_attr(ef_pad, we, be):
    return pl.pallas_call(
        _ea_body,
        grid=(_EGRID,),
        in_specs=[
            pl.BlockSpec((_EBLK, D_EDGE), lambda i: (i, 0)),
            pl.BlockSpec((D_EDGE, HID), lambda i: (0, 0)),
            pl.BlockSpec((1, HID), lambda i: (0, 0)),
        ],
        out_specs=pl.BlockSpec((_EBLK, HID), lambda i: (i, 0)),
        out_shape=jax.ShapeDtypeStruct((EP, HID), F32),
    )(ef_pad, we, be)


def _layer_body(g2_ref, e_ref, x_ref, w_ref, b_ref, sc_ref, sh_ref, out_ref):
    g = g2_ref[0] + g2_ref[1] + e_ref[...]
    h = jnp.dot(g, w_ref[...], preferred_element_type=F32) + b_ref[...]
    out_ref[...] = x_ref[...] + jnp.maximum(h, 0.0) * sc_ref[...] + sh_ref[...]


def _tc_layer(g2, eagg, x, w, b, scale, shift):
    return pl.pallas_call(
        _layer_body,
        grid=(_GRID,),
        in_specs=[
            pl.BlockSpec((2, _BLK, HID), lambda i: (0, i, 0)),
            pl.BlockSpec((_BLK, HID), lambda i: (i, 0)),
            pl.BlockSpec((_BLK, HID), lambda i: (i, 0)),
            pl.BlockSpec((HID, HID), lambda i: (0, 0)),
            pl.BlockSpec((1, HID), lambda i: (0, 0)),
            pl.BlockSpec((1, HID), lambda i: (0, 0)),
            pl.BlockSpec((1, HID), lambda i: (0, 0)),
        ],
        out_specs=pl.BlockSpec((_BLK, HID), lambda i: (i, 0)),
        out_shape=jax.ShapeDtypeStruct((NP, HID), F32),
    )(g2, eagg, x, w, b, scale, shift)


def _head_body(x_ref, wd_ref, bd_ref, sc_ref, sh_ref, wf_ref, bf_ref, out_ref):
    rows = lax.broadcasted_iota(jnp.int32, (NP, 1), 0)
    mask = (rows < N_NODES).astype(F32)
    m = jnp.sum(x_ref[...] * mask, axis=0, keepdims=True) / float(N_NODES)
    for i in range(NUM_DENSE):
        h = jnp.dot(m, wd_ref[i], preferred_element_type=F32) + bd_ref[i]
        m = jnp.maximum(h, 0.0) * sc_ref[i] + sh_ref[i]
    out_ref[...] = jnp.dot(m, wf_ref[...], preferred_element_type=F32) + bf_ref[...]


def _tc_head(x, wd, bd, scale, shift, wf, bf):
    return pl.pallas_call(
        _head_body,
        in_specs=[
            pl.BlockSpec((NP, HID), lambda: (0, 0)),
            pl.BlockSpec((NUM_DENSE, HID, HID), lambda: (0, 0, 0)),
            pl.BlockSpec((NUM_DENSE, 1, HID), lambda: (0, 0, 0)),
            pl.BlockSpec((NUM_DENSE, 1, HID), lambda: (0, 0, 0)),
            pl.BlockSpec((NUM_DENSE, 1, HID), lambda: (0, 0, 0)),
            pl.BlockSpec((HID, HID), lambda: (0, 0)),
            pl.BlockSpec((1, HID), lambda: (0, 0)),
        ],
        out_specs=pl.BlockSpec((1, HID), lambda: (0, 0)),
        out_shape=jax.ShapeDtypeStruct((1, HID), F32),
    )(x, wd, bd, scale, shift, wf, bf)


def kernel(node_feat, edge_index, edge_feat, params):
    p = params
    row = edge_index[0]
    col = edge_index[1]
    pad = EP - N_EDGES
    rowp = jnp.concatenate([row, jnp.full((pad,), N_NODES, jnp.int32)])
    colp = jnp.concatenate([col, jnp.zeros((pad,), jnp.int32)])
    rowp = rowp.reshape(NW, CHUNKS_PER_W, CHUNK)
    colp = colp.reshape(NW, CHUNKS_PER_W, CHUNK)

    ef_pad = jnp.concatenate([edge_feat, jnp.zeros((pad, D_EDGE), F32)], 0)
    nf_pad = jnp.concatenate([node_feat, jnp.zeros((NP - N_NODES, D_FEAT), F32)], 0)
    zeros128 = jnp.zeros((ROWS_PER_TILE, HID), F32)

    ea_pad = _tc_edge_attr(ef_pad, p["W_edge"], p["b_edge"].reshape(1, HID))
    ef2 = _sc_edge_scatter(ea_pad, rowp, zeros128)
    x, eagg = _tc_project(nf_pad, p["W_node"], p["b_node"].reshape(1, HID), ef2)

    for i in range(NUM_CONV):
        g2 = _sc_gather_scatter(x, colp, rowp, zeros128)
        scale = p["g_conv%d" % i] * lax.rsqrt(p["mv_conv%d" % i] + EPS)
        shift = p["be_conv%d" % i] - p["mm_conv%d" % i] * scale
        x = _tc_layer(g2, eagg, x, p["W_conv%d" % i],
                      p["b_conv%d" % i].reshape(1, HID),
                      scale.reshape(1, HID), shift.reshape(1, HID))

    wd = jnp.stack([p["W_out%d" % i] for i in range(NUM_DENSE)])
    bd = jnp.stack([p["b_out%d" % i].reshape(1, HID) for i in range(NUM_DENSE)])
    scs, shs = [], []
    for i in range(NUM_DENSE):
        sc = p["g_out%d" % i] * lax.rsqrt(p["mv_out%d" % i] + EPS)
        scs.append(sc.reshape(1, HID))
        shs.append((p["be_out%d" % i] - p["mm_out%d" % i] * sc).reshape(1, HID))
    wf = jnp.concatenate([p["W_final"], jnp.zeros((HID, HID - 3), F32)], 1)
    bf = jnp.concatenate([p["b_final"], jnp.zeros((HID - 3,), F32)]).reshape(1, HID)

    out = _tc_head(x, wd, bd, jnp.stack(scs), jnp.stack(shs), wf, bf)
    return out[0, :3]


# speedup vs baseline: 4.4346x; 1.2146x over previous
import functools

import jax
import jax.numpy as jnp
from jax import lax
from jax.experimental import pallas as pl
from jax.experimental.pallas import tpu as pltpu
from jax.experimental.pallas import tpu_sc as plsc

F32 = jnp.float32

N_NODES = 10000
N_EDGES = 320000
D_FEAT = 128
HID = 128
D_EDGE = 16
NUM_CONV = 4
NUM_DENSE = 3
EPS = 1e-3

NW = 32
CHUNK = 128
CHUNKS_PER_W = (N_EDGES + NW * CHUNK - 1) // (NW * CHUNK)
EDGES_PER_W = CHUNKS_PER_W * CHUNK
EP = NW * EDGES_PER_W
NP = 10112
ROWS_PER_TILE = NP // 16
_PH = (CHUNKS_PER_W + 1) // 2

_sc_mesh = plsc.VectorSubcoreMesh(core_axis_name="c", subcore_axis_name="s")


@functools.partial(
    pl.kernel,
    out_type=jax.ShapeDtypeStruct((2, NP, HID), F32),
    mesh=_sc_mesh,
    scratch_types=[
        pltpu.VMEM((CHUNKS_PER_W, CHUNK), jnp.int32),
        pltpu.VMEM((CHUNK, HID), F32),
        pltpu.VMEM((CHUNK, HID), F32),
        pltpu.VMEM_SHARED((NP, HID), F32),
        pltpu.SemaphoreType.DMA,
        pltpu.SemaphoreType.DMA,
    ],
)
def _sc_edge_scatter(ef_hbm, rowp_hbm, zeros_hbm, out_hbm,
                     row_slab, dbuf_a, dbuf_b, acc, sem_a, sem_b):
    c = lax.axis_index("c")
    s = lax.axis_index("s")
    w = c * 16 + s
    pltpu.sync_copy(rowp_hbm.at[w], row_slab)
    pltpu.sync_copy(zeros_hbm, acc.at[pl.ds(s * ROWS_PER_TILE, ROWS_PER_TILE)])
    plsc.subcore_barrier()

    def _src(j):
        return ef_hbm.at[pl.ds(w * EDGES_PER_W + j * CHUNK, CHUNK)]

    PAIRS = (CHUNKS_PER_W - 1) // 2
    pltpu.async_copy(_src(0), dbuf_a, sem_a)

    def body(i, _):
        j = 2 * i
        pltpu.async_copy(_src(j + 1), dbuf_b, sem_b)
        pltpu.make_async_copy(_src(j), dbuf_a, sem_a).wait()
        pltpu.sync_copy(dbuf_a, acc.at[row_slab.at[j]], add=True)
        pltpu.async_copy(_src(j + 2), dbuf_a, sem_a)
        pltpu.make_async_copy(_src(j + 1), dbuf_b, sem_b).wait()
        pltpu.sync_copy(dbuf_b, acc.at[row_slab.at[j + 1]], add=True)
        return _

    lax.fori_loop(0, PAIRS, body, None)
    j_last = CHUNKS_PER_W - 1
    pltpu.make_async_copy(_src(j_last), dbuf_a, sem_a).wait()
    pltpu.sync_copy(dbuf_a, acc.at[row_slab.at[j_last]], add=True)
    plsc.subcore_barrier()
    pltpu.sync_copy(acc.at[pl.ds(s * ROWS_PER_TILE, ROWS_PER_TILE)],
                    out_hbm.at[c, pl.ds(s * ROWS_PER_TILE, ROWS_PER_TILE)])


@functools.partial(
    pl.kernel,
    out_type=jax.ShapeDtypeStruct((2, NP, HID), F32),
    mesh=_sc_mesh,
    scratch_types=[
        pltpu.VMEM((_PH, CHUNK), jnp.int32),
        pltpu.VMEM((_PH, CHUNK), jnp.int32),
        pltpu.VMEM((CHUNK, HID), F32),
        pltpu.VMEM((CHUNK, HID), F32),
        pltpu.VMEM_SHARED((NP, HID), F32),
        pltpu.SemaphoreType.DMA,
        pltpu.SemaphoreType.DMA,
    ],
)
def _sc_gather_scatter(x_hbm, colp_hbm, rowp_hbm, zeros_hbm, out_hbm,
                       col_slab, row_slab, gbuf_a, gbuf_b, acc, sem_a, sem_b):
    c = lax.axis_index("c")
    s = lax.axis_index("s")
    w = c * 16 + s
    pltpu.sync_copy(zeros_hbm, acc.at[pl.ds(s * ROWS_PER_TILE, ROWS_PER_TILE)])
    plsc.subcore_barrier()

    for p in range(2):
        n = _PH if p == 0 else CHUNKS_PER_W - _PH
        pltpu.sync_copy(colp_hbm.at[w, pl.ds(p * _PH, n)], col_slab.at[pl.ds(0, n)])
        pltpu.sync_copy(rowp_hbm.at[w, pl.ds(p * _PH, n)], row_slab.at[pl.ds(0, n)])
        pltpu.async_copy(x_hbm.at[col_slab.at[0]], gbuf_a, sem_a)

        def body(i, _):
            j = 2 * i
            pltpu.async_copy(x_hbm.at[col_slab.at[j + 1]], gbuf_b, sem_b)
            pltpu.make_async_copy(x_hbm.at[col_slab.at[j]], gbuf_a, sem_a).wait()
            pltpu.sync_copy(gbuf_a, acc.at[row_slab.at[j]], add=True)

            @pl.when(j + 2 < n)
            def _():
                pltpu.async_copy(x_hbm.at[col_slab.at[j + 2]], gbuf_a, sem_a)

            pltpu.make_async_copy(x_hbm.at[col_slab.at[j + 1]], gbuf_b, sem_b).wait()
            pltpu.sync_copy(gbuf_b, acc.at[row_slab.at[j + 1]], add=True)
            return _

        lax.fori_loop(0, n // 2, body, None)
        if n % 2:
            j_last = n - 1
            pltpu.make_async_copy(x_hbm.at[col_slab.at[j_last]], gbuf_a, sem_a).wait()
            pltpu.sync_copy(gbuf_a, acc.at[row_slab.at[j_last]], add=True)

    plsc.subcore_barrier()
    pltpu.sync_copy(acc.at[pl.ds(s * ROWS_PER_TILE, ROWS_PER_TILE)],
                    out_hbm.at[c, pl.ds(s * ROWS_PER_TILE, ROWS_PER_TILE)])


_BLK = 1264
_GRID = NP // _BLK


def _proj_body(nf_ref, wn_ref, bn_ref, ef2_ref, x0_ref, eagg_ref):
    x0_ref[...] = (jnp.dot(nf_ref[...], wn_ref[...], preferred_element_type=F32)
                   + bn_ref[...])
    eagg_ref[...] = ef2_ref[0] + ef2_ref[1]


def _tc_project(nf_pad, wn, bn, ef2):
    return pl.pallas_call(
        _proj_body,
        grid=(_GRID,),
        in_specs=[
            pl.BlockSpec((_BLK, D_FEAT), lambda i: (i, 0)),
            pl.BlockSpec((D_FEAT, HID), lambda i: (0, 0)),
            pl.BlockSpec((1, HID), lambda i: (0, 0)),
            pl.BlockSpec((2, _BLK, HID), lambda i: (0, i, 0)),
        ],
        out_specs=[
            pl.BlockSpec((_BLK, HID), lambda i: (i, 0)),
            pl.BlockSpec((_BLK, HID), lambda i: (i, 0)),
        ],
        out_shape=[
            jax.ShapeDtypeStruct((NP, HID), F32),
            jax.ShapeDtypeStruct((NP, HID), F32),
        ],
    )(nf_pad, wn, bn, ef2)


_EBLK = 4096
_EGRID = EP // _EBLK


def _ea_body(ef_ref, we_ref, be_ref, out_ref):
    out_ref[...] = (jnp.dot(ef_ref[...], we_ref[...], preferred_element_type=F32)
                    + be_ref[...])


def _tc_edge_attr(ef_pad, we, be):
    return pl.pallas_call(
        _ea_body,
        grid=(_EGRID,),
        in_specs=[
            pl.BlockSpec((_EBLK, D_EDGE), lambda i: (i, 0)),
            pl.BlockSpec((D_EDGE, HID), lambda i: (0, 0)),
            pl.BlockSpec((1, HID), lambda i: (0, 0)),
        ],
        out_specs=pl.BlockSpec((_EBLK, HID), lambda i: (i, 0)),
        out_shape=jax.ShapeDtypeStruct((EP, HID), F32),
    )(ef_pad, we, be)


def _layer_body(g2_ref, e_ref, x_ref, w_ref, b_ref, sc_ref, sh_ref, out_ref):
    g = g2_ref[0] + g2_ref[1] + e_ref[...]
    h = jnp.dot(g, w_ref[...], preferred_element_type=F32) + b_ref[...]
    out_ref[...] = x_ref[...] + jnp.maximum(h, 0.0) * sc_ref[...] + sh_ref[...]


def _tc_layer(g2, eagg, x, w, b, scale, shift):
    return pl.pallas_call(
        _layer_body,
        grid=(_GRID,),
        in_specs=[
            pl.BlockSpec((2, _BLK, HID), lambda i: (0, i, 0)),
            pl.BlockSpec((_BLK, HID), lambda i: (i, 0)),
            pl.BlockSpec((_BLK, HID), lambda i: (i, 0)),
            pl.BlockSpec((HID, HID), lambda i: (0, 0)),
            pl.BlockSpec((1, HID), lambda i: (0, 0)),
            pl.BlockSpec((1, HID), lambda i: (0, 0)),
            pl.BlockSpec((1, HID), lambda i: (0, 0)),
        ],
        out_specs=pl.BlockSpec((_BLK, HID), lambda i: (i, 0)),
        out_shape=jax.ShapeDtypeStruct((NP, HID), F32),
    )(g2, eagg, x, w, b, scale, shift)


def _head_body(x_ref, wd_ref, bd_ref, sc_ref, sh_ref, wf_ref, bf_ref, out_ref):
    rows = lax.broadcasted_iota(jnp.int32, (NP, 1), 0)
    mask = (rows < N_NODES).astype(F32)
    m = jnp.sum(x_ref[...] * mask, axis=0, keepdims=True) / float(N_NODES)
    for i in range(NUM_DENSE):
        h = jnp.dot(m, wd_ref[i], preferred_element_type=F32) + bd_ref[i]
        m = jnp.maximum(h, 0.0) * sc_ref[i] + sh_ref[i]
    out_ref[...] = jnp.dot(m, wf_ref[...], preferred_element_type=F32) + bf_ref[...]


def _tc_head(x, wd, bd, scale, shift, wf, bf):
    return pl.pallas_call(
        _head_body,
        in_specs=[
            pl.BlockSpec((NP, HID), lambda: (0, 0)),
            pl.BlockSpec((NUM_DENSE, HID, HID), lambda: (0, 0, 0)),
            pl.BlockSpec((NUM_DENSE, 1, HID), lambda: (0, 0, 0)),
            pl.BlockSpec((NUM_DENSE, 1, HID), lambda: (0, 0, 0)),
            pl.BlockSpec((NUM_DENSE, 1, HID), lambda: (0, 0, 0)),
            pl.BlockSpec((HID, HID), lambda: (0, 0)),
            pl.BlockSpec((1, HID), lambda: (0, 0)),
        ],
        out_specs=pl.BlockSpec((1, HID), lambda: (0, 0)),
        out_shape=jax.ShapeDtypeStruct((1, HID), F32),
    )(x, wd, bd, scale, shift, wf, bf)


def kernel(node_feat, edge_index, edge_feat, params):
    p = params
    row = edge_index[0]
    col = edge_index[1]
    pad = EP - N_EDGES
    rowp = jnp.concatenate([row, jnp.full((pad,), N_NODES, jnp.int32)])
    colp = jnp.concatenate([col, jnp.zeros((pad,), jnp.int32)])
    rowp = rowp.reshape(NW, CHUNKS_PER_W, CHUNK)
    colp = colp.reshape(NW, CHUNKS_PER_W, CHUNK)

    ef_pad = jnp.concatenate([edge_feat, jnp.zeros((pad, D_EDGE), F32)], 0)
    nf_pad = jnp.concatenate([node_feat, jnp.zeros((NP - N_NODES, D_FEAT), F32)], 0)
    zeros128 = jnp.zeros((ROWS_PER_TILE, HID), F32)

    ea_pad = _tc_edge_attr(ef_pad, p["W_edge"], p["b_edge"].reshape(1, HID))
    ef2 = _sc_edge_scatter(ea_pad, rowp, zeros128)
    x, eagg = _tc_project(nf_pad, p["W_node"], p["b_node"].reshape(1, HID), ef2)

    for i in range(NUM_CONV):
        g2 = _sc_gather_scatter(x, colp, rowp, zeros128)
        scale = p["g_conv%d" % i] * lax.rsqrt(p["mv_conv%d" % i] + EPS)
        shift = p["be_conv%d" % i] - p["mm_conv%d" % i] * scale
        x = _tc_layer(g2, eagg, x, p["W_conv%d" % i],
                      p["b_conv%d" % i].reshape(1, HID),
                      scale.reshape(1, HID), shift.reshape(1, HID))

    wd = jnp.stack([p["W_out%d" % i] for i in range(NUM_DENSE)])
    bd = jnp.stack([p["b_out%d" % i].reshape(1, HID) for i in range(NUM_DENSE)])
    scs, shs = [], []
    for i in range(NUM_DENSE):
        sc = p["g_out%d" % i] * lax.rsqrt(p["mv_out%d" % i] + EPS)
        scs.append(sc.reshape(1, HID))
        shs.append((p["be_out%d" % i] - p["mm_out%d" % i] * sc).reshape(1, HID))
    wf = jnp.concatenate([p["W_final"], jnp.zeros((HID, HID - 3), F32)], 1)
    bf = jnp.concatenate([p["b_final"], jnp.zeros((HID - 3,), F32)]).reshape(1, HID)

    out = _tc_head(x, wd, bd, jnp.stack(scs), jnp.stack(shs), wf, bf)
    return out[0, :3]


# speedup vs baseline: 4.7814x; 1.0782x over previous
import functools

import jax
import jax.numpy as jnp
from jax import lax
from jax.experimental import pallas as pl
from jax.experimental.pallas import tpu as pltpu
from jax.experimental.pallas import tpu_sc as plsc

F32 = jnp.float32

N_NODES = 10000
N_EDGES = 320000
D_FEAT = 128
HID = 128
D_EDGE = 16
NUM_CONV = 4
NUM_DENSE = 3
EPS = 1e-3

NW = 32
CHUNK = 128
CHUNKS_PER_W = (N_EDGES + NW * CHUNK - 1) // (NW * CHUNK)
EDGES_PER_W = CHUNKS_PER_W * CHUNK
EP = NW * EDGES_PER_W
NP = 10112
ROWS_PER_TILE = NP // 16
_PH = (CHUNKS_PER_W + 1) // 2
CH0 = 109
CH1 = 2 * CHUNKS_PER_W - CH0

_sc_mesh = plsc.VectorSubcoreMesh(core_axis_name="c", subcore_axis_name="s")


@functools.partial(
    pl.kernel,
    out_type=jax.ShapeDtypeStruct((2, NP, HID), F32),
    mesh=_sc_mesh,
    scratch_types=[
        pltpu.VMEM((CHUNKS_PER_W, CHUNK), jnp.int32),
        pltpu.VMEM((CHUNK, HID), F32),
        pltpu.VMEM((CHUNK, HID), F32),
        pltpu.VMEM_SHARED((NP, HID), F32),
        pltpu.SemaphoreType.DMA,
        pltpu.SemaphoreType.DMA,
    ],
)
def _sc_edge_scatter(ef_hbm, rowp_hbm, zeros_hbm, out_hbm,
                     row_slab, dbuf_a, dbuf_b, acc, sem_a, sem_b):
    c = lax.axis_index("c")
    s = lax.axis_index("s")
    w = c * 16 + s
    pltpu.sync_copy(rowp_hbm.at[w], row_slab)
    pltpu.sync_copy(zeros_hbm, acc.at[pl.ds(s * ROWS_PER_TILE, ROWS_PER_TILE)])
    plsc.subcore_barrier()

    def _src(j):
        return ef_hbm.at[pl.ds(w * EDGES_PER_W + j * CHUNK, CHUNK)]

    PAIRS = (CHUNKS_PER_W - 1) // 2
    pltpu.async_copy(_src(0), dbuf_a, sem_a)

    def body(i, _):
        j = 2 * i
        pltpu.async_copy(_src(j + 1), dbuf_b, sem_b)
        pltpu.make_async_copy(_src(j), dbuf_a, sem_a).wait()
        pltpu.sync_copy(dbuf_a, acc.at[row_slab.at[j]], add=True)
        pltpu.async_copy(_src(j + 2), dbuf_a, sem_a)
        pltpu.make_async_copy(_src(j + 1), dbuf_b, sem_b).wait()
        pltpu.sync_copy(dbuf_b, acc.at[row_slab.at[j + 1]], add=True)
        return _

    lax.fori_loop(0, PAIRS, body, None)
    j_last = CHUNKS_PER_W - 1
    pltpu.make_async_copy(_src(j_last), dbuf_a, sem_a).wait()
    pltpu.sync_copy(dbuf_a, acc.at[row_slab.at[j_last]], add=True)
    plsc.subcore_barrier()
    pltpu.sync_copy(acc.at[pl.ds(s * ROWS_PER_TILE, ROWS_PER_TILE)],
                    out_hbm.at[c, pl.ds(s * ROWS_PER_TILE, ROWS_PER_TILE)])


def _agg_pipeline(x_hbm, colp, rowp, s, nch,
                  col_slab, row_slab, gbuf_a, gbuf_b, acc, sem_a, sem_b):
    for base in range(0, nch, _PH):
        n = min(_PH, nch - base)
        pltpu.sync_copy(colp.at[s, pl.ds(base, n)], col_slab.at[pl.ds(0, n)])
        pltpu.sync_copy(rowp.at[s, pl.ds(base, n)], row_slab.at[pl.ds(0, n)])
        pltpu.async_copy(x_hbm.at[col_slab.at[0]], gbuf_a, sem_a)

        def body(i, _):
            j = 2 * i
            pltpu.async_copy(x_hbm.at[col_slab.at[j + 1]], gbuf_b, sem_b)
            pltpu.make_async_copy(x_hbm.at[col_slab.at[j]], gbuf_a, sem_a).wait()
            pltpu.sync_copy(gbuf_a, acc.at[row_slab.at[j]], add=True)

            @pl.when(j + 2 < n)
            def _():
                pltpu.async_copy(x_hbm.at[col_slab.at[j + 2]], gbuf_a, sem_a)

            pltpu.make_async_copy(x_hbm.at[col_slab.at[j + 1]], gbuf_b, sem_b).wait()
            pltpu.sync_copy(gbuf_b, acc.at[row_slab.at[j + 1]], add=True)
            return _

        lax.fori_loop(0, n // 2, body, None)
        if n % 2:
            j_last = n - 1
            pltpu.make_async_copy(x_hbm.at[col_slab.at[j_last]], gbuf_a, sem_a).wait()
            pltpu.sync_copy(gbuf_a, acc.at[row_slab.at[j_last]], add=True)


@functools.partial(
    pl.kernel,
    out_type=jax.ShapeDtypeStruct((2, NP, HID), F32),
    mesh=_sc_mesh,
    scratch_types=[
        pltpu.VMEM((_PH, CHUNK), jnp.int32),
        pltpu.VMEM((_PH, CHUNK), jnp.int32),
        pltpu.VMEM((CHUNK, HID), F32),
        pltpu.VMEM((CHUNK, HID), F32),
        pltpu.VMEM_SHARED((NP, HID), F32),
        pltpu.SemaphoreType.DMA,
        pltpu.SemaphoreType.DMA,
    ],
)
def _sc_gather_scatter(x_hbm, colp0_hbm, rowp0_hbm, colp1_hbm, rowp1_hbm,
                       zeros_hbm, out_hbm,
                       col_slab, row_slab, gbuf_a, gbuf_b, acc, sem_a, sem_b):
    c = lax.axis_index("c")
    s = lax.axis_index("s")
    pltpu.sync_copy(zeros_hbm, acc.at[pl.ds(s * ROWS_PER_TILE, ROWS_PER_TILE)])
    plsc.subcore_barrier()

    @pl.when(c == 0)
    def _():
        _agg_pipeline(x_hbm, colp0_hbm, rowp0_hbm, s, CH0,
                      col_slab, row_slab, gbuf_a, gbuf_b, acc, sem_a, sem_b)

    @pl.when(c == 1)
    def _():
        _agg_pipeline(x_hbm, colp1_hbm, rowp1_hbm, s, CH1,
                      col_slab, row_slab, gbuf_a, gbuf_b, acc, sem_a, sem_b)

    plsc.subcore_barrier()
    pltpu.sync_copy(acc.at[pl.ds(s * ROWS_PER_TILE, ROWS_PER_TILE)],
                    out_hbm.at[c, pl.ds(s * ROWS_PER_TILE, ROWS_PER_TILE)])


_BLK = 1264
_GRID = NP // _BLK


def _proj_body(nf_ref, wn_ref, bn_ref, ef2_ref, x0_ref, eagg_ref):
    x0_ref[...] = (jnp.dot(nf_ref[...], wn_ref[...], preferred_element_type=F32)
                   + bn_ref[...])
    eagg_ref[...] = ef2_ref[0] + ef2_ref[1]


def _tc_project(nf_pad, wn, bn, ef2):
    return pl.pallas_call(
        _proj_body,
        grid=(_GRID,),
        in_specs=[
            pl.BlockSpec((_BLK, D_FEAT), lambda i: (i, 0)),
            pl.BlockSpec((D_FEAT, HID), lambda i: (0, 0)),
            pl.BlockSpec((1, HID), lambda i: (0, 0)),
            pl.BlockSpec((2, _BLK, HID), lambda i: (0, i, 0)),
        ],
        out_specs=[
            pl.BlockSpec((_BLK, HID), lambda i: (i, 0)),
            pl.BlockSpec((_BLK, HID), lambda i: (i, 0)),
        ],
        out_shape=[
            jax.ShapeDtypeStruct((NP, HID), F32),
            jax.ShapeDtypeStruct((NP, HID), F32),
        ],
    )(nf_pad, wn, bn, ef2)


_EBLK = 4096
_EGRID = EP // _EBLK


def _ea_body(ef_ref, we_ref, be_ref, out_ref):
    out_ref[...] = (jnp.dot(ef_ref[...], we_ref[...], preferred_element_type=F32)
                    + be_ref[...])


def _tc_edge_attr(ef_pad, we, be):
    return pl.pallas_call(
        _ea_body,
        grid=(_EGRID,),
        in_specs=[
            pl.BlockSpec((_EBLK, D_EDGE), lambda i: (i, 0)),
            pl.BlockSpec((D_EDGE, HID), lambda i: (0, 0)),
            pl.BlockSpec((1, HID), lambda i: (0, 0)),
        ],
        out_specs=pl.BlockSpec((_EBLK, HID), lambda i: (i, 0)),
        out_shape=jax.ShapeDtypeStruct((EP, HID), F32),
    )(ef_pad, we, be)


def _layer_body(g2_ref, e_ref, x_ref, w_ref, b_ref, sc_ref, sh_ref, out_ref):
    g = g2_ref[0] + g2_ref[1] + e_ref[...]
    h = jnp.dot(g, w_ref[...], preferred_element_type=F32) + b_ref[...]
    out_ref[...] = x_ref[...] + jnp.maximum(h, 0.0) * sc_ref[...] + sh_ref[...]


def _tc_layer(g2, eagg, x, w, b, scale, shift):
    return pl.pallas_call(
        _layer_body,
        grid=(_GRID,),
        in_specs=[
            pl.BlockSpec((2, _BLK, HID), lambda i: (0, i, 0)),
            pl.BlockSpec((_BLK, HID), lambda i: (i, 0)),
            pl.BlockSpec((_BLK, HID), lambda i: (i, 0)),
            pl.BlockSpec((HID, HID), lambda i: (0, 0)),
            pl.BlockSpec((1, HID), lambda i: (0, 0)),
            pl.BlockSpec((1, HID), lambda i: (0, 0)),
            pl.BlockSpec((1, HID), lambda i: (0, 0)),
        ],
        out_specs=pl.BlockSpec((_BLK, HID), lambda i: (i, 0)),
        out_shape=jax.ShapeDtypeStruct((NP, HID), F32),
    )(g2, eagg, x, w, b, scale, shift)


def _head_body(x_ref, wd_ref, bd_ref, sc_ref, sh_ref, wf_ref, bf_ref, out_ref):
    rows = lax.broadcasted_iota(jnp.int32, (NP, 1), 0)
    mask = (rows < N_NODES).astype(F32)
    m = jnp.sum(x_ref[...] * mask, axis=0, keepdims=True) / float(N_NODES)
    for i in range(NUM_DENSE):
        h = jnp.dot(m, wd_ref[i], preferred_element_type=F32) + bd_ref[i]
        m = jnp.maximum(h, 0.0) * sc_ref[i] + sh_ref[i]
    out_ref[...] = jnp.dot(m, wf_ref[...], preferred_element_type=F32) + bf_ref[...]


def _tc_head(x, wd, bd, scale, shift, wf, bf):
    return pl.pallas_call(
        _head_body,
        in_specs=[
            pl.BlockSpec((NP, HID), lambda: (0, 0)),
            pl.BlockSpec((NUM_DENSE, HID, HID), lambda: (0, 0, 0)),
            pl.BlockSpec((NUM_DENSE, 1, HID), lambda: (0, 0, 0)),
            pl.BlockSpec((NUM_DENSE, 1, HID), lambda: (0, 0, 0)),
            pl.BlockSpec((NUM_DENSE, 1, HID), lambda: (0, 0, 0)),
            pl.BlockSpec((HID, HID), lambda: (0, 0)),
            pl.BlockSpec((1, HID), lambda: (0, 0)),
        ],
        out_specs=pl.BlockSpec((1, HID), lambda: (0, 0)),
        out_shape=jax.ShapeDtypeStruct((1, HID), F32),
    )(x, wd, bd, scale, shift, wf, bf)


def kernel(node_feat, edge_index, edge_feat, params):
    p = params
    row = edge_index[0]
    col = edge_index[1]
    pad = EP - N_EDGES
    rowp_flat = jnp.concatenate([row, jnp.full((pad,), N_NODES, jnp.int32)])
    colp_flat = jnp.concatenate([col, jnp.zeros((pad,), jnp.int32)])
    rowp = rowp_flat.reshape(NW, CHUNKS_PER_W, CHUNK)
    n0 = 16 * CH0 * CHUNK
    colp0 = colp_flat[:n0].reshape(16, CH0, CHUNK)
    rowp0 = rowp_flat[:n0].reshape(16, CH0, CHUNK)
    colp1 = colp_flat[n0:].reshape(16, CH1, CHUNK)
    rowp1 = rowp_flat[n0:].reshape(16, CH1, CHUNK)

    ef_pad = jnp.concatenate([edge_feat, jnp.zeros((pad, D_EDGE), F32)], 0)
    nf_pad = jnp.concatenate([node_feat, jnp.zeros((NP - N_NODES, D_FEAT), F32)], 0)
    zeros128 = jnp.zeros((ROWS_PER_TILE, HID), F32)

    ea_pad = _tc_edge_attr(ef_pad, p["W_edge"], p["b_edge"].reshape(1, HID))
    ef2 = _sc_edge_scatter(ea_pad, rowp, zeros128)
    x, eagg = _tc_project(nf_pad, p["W_node"], p["b_node"].reshape(1, HID), ef2)

    for i in range(NUM_CONV):
        g2 = _sc_gather_scatter(x, colp0, rowp0, colp1, rowp1, zeros128)
        scale = p["g_conv%d" % i] * lax.rsqrt(p["mv_conv%d" % i] + EPS)
        shift = p["be_conv%d" % i] - p["mm_conv%d" % i] * scale
        x = _tc_layer(g2, eagg, x, p["W_conv%d" % i],
                      p["b_conv%d" % i].reshape(1, HID),
                      scale.reshape(1, HID), shift.reshape(1, HID))

    wd = jnp.stack([p["W_out%d" % i] for i in range(NUM_DENSE)])
    bd = jnp.stack([p["b_out%d" % i].reshape(1, HID) for i in range(NUM_DENSE)])
    scs, shs = [], []
    for i in range(NUM_DENSE):
        sc = p["g_out%d" % i] * lax.rsqrt(p["mv_out%d" % i] + EPS)
        scs.append(sc.reshape(1, HID))
        shs.append((p["be_out%d" % i] - p["mm_out%d" % i] * sc).reshape(1, HID))
    wf = jnp.concatenate([p["W_final"], jnp.zeros((HID, HID - 3), F32)], 1)
    bf = jnp.concatenate([p["b_final"], jnp.zeros((HID - 3,), F32)]).reshape(1, HID)

    out = _tc_head(x, wd, bd, jnp.stack(scs), jnp.stack(shs), wf, bf)
    return out[0, :3]
